# baseline (speedup 1.0000x reference)
import functools

import jax
import jax.numpy as jnp
from jax.experimental import pallas as pl
from jax.experimental.pallas import tpu as pltpu


def _argmin_codes(dist, k_sz, split_merge):
    if split_merge:
        h = k_sz // 2
        d0 = dist[:, :h]
        d1 = dist[:, h:]
        m0 = jnp.min(d0, axis=1, keepdims=True)
        m1 = jnp.min(d1, axis=1, keepdims=True)
        il = jax.lax.broadcasted_iota(jnp.int32, d0.shape, 1)
        a0 = jnp.min(jnp.where(d0 == m0, il, h), axis=1)
        a1 = jnp.min(jnp.where(d1 == m1, il, h), axis=1) + h
        m0r = m0[:, 0].astype(jnp.bfloat16).astype(jnp.float32)
        return jnp.where(m1[:, 0] < m0r, a1, a0)
    m = jnp.min(dist, axis=1, keepdims=True)
    iota = jax.lax.broadcasted_iota(jnp.int32, dist.shape, 1)
    return jnp.min(jnp.where(dist == m, iota, k_sz), axis=1)


def _stage_body(x2_ref, res_ref, zq_ref, cb_ref, e2_ref,
                codes_ref, usage_ref, res_out, zq_out,
                *, nb, k_sz, inv_b, split_merge):
    b = pl.program_id(0)
    x = res_ref[...]
    cb = cb_ref[...]
    xe = jax.lax.dot_general(
        x, cb, dimension_numbers=(((1,), (1,)), ((), ())),
        preferred_element_type=jnp.float32)
    dist = x2_ref[...] + e2_ref[...] - 2.0 * xe
    codes = _argmin_codes(dist, k_sz, split_merge)
    iota = jax.lax.broadcasted_iota(jnp.int32, dist.shape, 1)
    onehot = (iota == codes[:, None]).astype(jnp.float32)
    quant = jax.lax.dot_general(
        onehot, cb, dimension_numbers=(((1,), (0,)), ((), ())),
        preferred_element_type=jnp.float32,
        precision=jax.lax.Precision.HIGHEST)
    counts = jnp.sum(onehot, axis=0, keepdims=True)

    codes_ref[...] = codes[:, None]
    res_out[...] = x - quant
    zq_out[...] = zq_ref[...] + quant

    @pl.when(b == 0)
    def _():
        usage_ref[...] = jnp.zeros_like(counts)

    usage_ref[...] += counts

    @pl.when(b == nb - 1)
    def _():
        usage_ref[...] = usage_ref[...] * inv_b


def _stage_body_last(x2_ref, res_ref, zq_ref, cb_ref, e2_ref, z_ref,
                     codes_ref, usage_ref, res_out, zq_out, loss_ref,
                     *, nb, k_sz, inv_b, inv_n, split_merge):
    b = pl.program_id(0)
    x = res_ref[...]
    cb = cb_ref[...]
    xe = jax.lax.dot_general(
        x, cb, dimension_numbers=(((1,), (1,)), ((), ())),
        preferred_element_type=jnp.float32)
    dist = x2_ref[...] + e2_ref[...] - 2.0 * xe
    codes = _argmin_codes(dist, k_sz, split_merge)
    iota = jax.lax.broadcasted_iota(jnp.int32, dist.shape, 1)
    onehot = (iota == codes[:, None]).astype(jnp.float32)
    quant = jax.lax.dot_general(
        onehot, cb, dimension_numbers=(((1,), (0,)), ((), ())),
        preferred_element_type=jnp.float32,
        precision=jax.lax.Precision.HIGHEST)
    counts = jnp.sum(onehot, axis=0, keepdims=True)

    codes_ref[...] = codes[:, None]
    new_res = x - quant
    new_zq = zq_ref[...] + quant
    res_out[...] = new_res
    zq_out[...] = new_zq

    @pl.when(b == 0)
    def _():
        usage_ref[...] = jnp.zeros_like(counts)
        loss_ref[0, 0] = 0.0

    usage_ref[...] += counts
    loss_ref[0, 0] += jnp.sum((new_zq - z_ref[...]) ** 2)

    @pl.when(b == nb - 1)
    def _():
        usage_ref[...] = usage_ref[...] * inv_b
        loss_ref[0, 0] = loss_ref[0, 0] * inv_n


def _run_stage(x2, residual, zq, cb, e2, z, is_last, split_merge, bt):
    b_sz, d = residual.shape
    k_sz = cb.shape[0]
    nb = b_sz // bt
    row_spec = pl.BlockSpec((bt, d), lambda b: (b, 0))
    in_specs = [
        pl.BlockSpec((bt, 1), lambda b: (b, 0)),
        row_spec,
        row_spec,
        pl.BlockSpec((k_sz, d), lambda b: (0, 0)),
        pl.BlockSpec((1, k_sz), lambda b: (0, 0)),
    ]
    out_shapes = [
        jax.ShapeDtypeStruct((b_sz, 1), jnp.int32),
        jax.ShapeDtypeStruct((1, k_sz), jnp.float32),
        jax.ShapeDtypeStruct((b_sz, d), jnp.float32),
        jax.ShapeDtypeStruct((b_sz, d), jnp.float32),
    ]
    out_specs = [
        pl.BlockSpec((bt, 1), lambda b: (b, 0)),
        pl.BlockSpec((1, k_sz), lambda b: (0, 0)),
        row_spec,
        row_spec,
    ]
    args = [x2, residual, zq, cb, e2]
    if is_last:
        in_specs.append(row_spec)
        args.append(z)
        out_shapes.append(jax.ShapeDtypeStruct((1, 1), jnp.float32))
        out_specs.append(pl.BlockSpec(memory_space=pltpu.SMEM))
        body = functools.partial(_stage_body_last, nb=nb, k_sz=k_sz,
                                 inv_b=1.0 / b_sz, inv_n=1.0 / (b_sz * d),
                                 split_merge=split_merge)
    else:
        body = functools.partial(_stage_body, nb=nb, k_sz=k_sz,
                                 inv_b=1.0 / b_sz, split_merge=split_merge)
    return pl.pallas_call(
        body,
        grid=(nb,),
        in_specs=in_specs,
        out_specs=out_specs,
        out_shape=out_shapes,
        compiler_params=pltpu.CompilerParams(
            dimension_semantics=("arbitrary",)),
    )(*args)


def kernel(z, embed):
    r, k_sz, d = embed.shape
    b_sz = z.shape[0]
    bt = min(256, b_sz)
    residual = z
    zq = jnp.zeros_like(z)
    codes_list, usage_list = [], []
    loss = None
    for s in range(r):
        cb = embed[s]
        x2 = jnp.sum(residual ** 2, axis=1, keepdims=True)
        e2 = jnp.sum(cb ** 2, axis=1)[None, :]
        is_last = s == r - 1
        outs = _run_stage(x2, residual, zq, cb, e2, z, is_last, s == 0, bt)
        if is_last:
            codes_s, usage_s, residual, zq, loss = outs
        else:
            codes_s, usage_s, residual, zq = outs
        codes_list.append(codes_s)
        usage_list.append(usage_s)
    codes = jnp.concatenate(codes_list, axis=1)
    usage = jnp.concatenate(usage_list, axis=0)
    recon_loss = loss.reshape(())
    return (codes, zq, residual, recon_loss, usage)


# speedup vs baseline: 1.0004x; 1.0004x over previous
import functools

import jax
import jax.numpy as jnp
from jax.experimental import pallas as pl
from jax.experimental.pallas import tpu as pltpu


def _argmin_codes(dist, k_sz, split_merge):
    if split_merge:
        h = k_sz // 2
        d0 = dist[:, :h]
        d1 = dist[:, h:]
        m0 = jnp.min(d0, axis=1, keepdims=True)
        m1 = jnp.min(d1, axis=1, keepdims=True)
        il = jax.lax.broadcasted_iota(jnp.int32, d0.shape, 1)
        a0 = jnp.min(jnp.where(d0 == m0, il, h), axis=1)
        a1 = jnp.min(jnp.where(d1 == m1, il, h), axis=1) + h
        m0r = m0[:, 0].astype(jnp.bfloat16).astype(jnp.float32)
        return jnp.where(m1[:, 0] < m0r, a1, a0)
    m = jnp.min(dist, axis=1, keepdims=True)
    iota = jax.lax.broadcasted_iota(jnp.int32, dist.shape, 1)
    return jnp.min(jnp.where(dist == m, iota, k_sz), axis=1)


def _stage_body(x2_ref, res_ref, zq_ref, cb_ref, e2_ref,
                codes_ref, usage_ref, res_out, zq_out,
                *, nb, k_sz, inv_b, split_merge):
    b = pl.program_id(0)
    x = res_ref[...]
    cb = cb_ref[...]
    xe = jax.lax.dot_general(
        x, cb, dimension_numbers=(((1,), (1,)), ((), ())),
        preferred_element_type=jnp.float32)
    dist = x2_ref[...] + e2_ref[...] - 2.0 * xe
    codes = _argmin_codes(dist, k_sz, split_merge)
    iota = jax.lax.broadcasted_iota(jnp.int32, dist.shape, 1)
    onehot = (iota == codes[:, None]).astype(jnp.float32)
    quant = jax.lax.dot_general(
        onehot, cb, dimension_numbers=(((1,), (0,)), ((), ())),
        precision=jax.lax.Precision.HIGHEST,
        preferred_element_type=jnp.float32)
    counts = jnp.sum(onehot, axis=0, keepdims=True)

    codes_ref[...] = codes[:, None]
    res_out[...] = x - quant
    zq_out[...] = zq_ref[...] + quant

    @pl.when(b == 0)
    def _():
        usage_ref[...] = jnp.zeros_like(counts)

    usage_ref[...] += counts

    @pl.when(b == nb - 1)
    def _():
        usage_ref[...] = usage_ref[...] * inv_b


def _stage_body_last(x2_ref, res_ref, zq_ref, cb_ref, e2_ref, z_ref,
                     codes_ref, usage_ref, res_out, zq_out, loss_ref,
                     *, nb, k_sz, inv_b, inv_n, split_merge):
    b = pl.program_id(0)
    x = res_ref[...]
    cb = cb_ref[...]
    xe = jax.lax.dot_general(
        x, cb, dimension_numbers=(((1,), (1,)), ((), ())),
        preferred_element_type=jnp.float32)
    dist = x2_ref[...] + e2_ref[...] - 2.0 * xe
    codes = _argmin_codes(dist, k_sz, split_merge)
    iota = jax.lax.broadcasted_iota(jnp.int32, dist.shape, 1)
    onehot = (iota == codes[:, None]).astype(jnp.float32)
    quant = jax.lax.dot_general(
        onehot, cb, dimension_numbers=(((1,), (0,)), ((), ())),
        precision=jax.lax.Precision.HIGHEST,
        preferred_element_type=jnp.float32)
    counts = jnp.sum(onehot, axis=0, keepdims=True)

    codes_ref[...] = codes[:, None]
    new_res = x - quant
    new_zq = zq_ref[...] + quant
    res_out[...] = new_res
    zq_out[...] = new_zq

    @pl.when(b == 0)
    def _():
        usage_ref[...] = jnp.zeros_like(counts)
        loss_ref[0, 0] = 0.0

    usage_ref[...] += counts
    loss_ref[0, 0] += jnp.sum((new_zq - z_ref[...]) ** 2)

    @pl.when(b == nb - 1)
    def _():
        usage_ref[...] = usage_ref[...] * inv_b
        loss_ref[0, 0] = loss_ref[0, 0] * inv_n


def _run_stage(x2, residual, zq, cb, e2, z, is_last, split_merge, bt):
    b_sz, d = residual.shape
    k_sz = cb.shape[0]
    nb = b_sz // bt
    row_spec = pl.BlockSpec((bt, d), lambda b: (b, 0))
    in_specs = [
        pl.BlockSpec((bt, 1), lambda b: (b, 0)),
        row_spec,
        row_spec,
        pl.BlockSpec((k_sz, d), lambda b: (0, 0)),
        pl.BlockSpec((1, k_sz), lambda b: (0, 0)),
    ]
    out_shapes = [
        jax.ShapeDtypeStruct((b_sz, 1), jnp.int32),
        jax.ShapeDtypeStruct((1, k_sz), jnp.float32),
        jax.ShapeDtypeStruct((b_sz, d), jnp.float32),
        jax.ShapeDtypeStruct((b_sz, d), jnp.float32),
    ]
    out_specs = [
        pl.BlockSpec((bt, 1), lambda b: (b, 0)),
        pl.BlockSpec((1, k_sz), lambda b: (0, 0)),
        row_spec,
        row_spec,
    ]
    args = [x2, residual, zq, cb, e2]
    if is_last:
        in_specs.append(row_spec)
        args.append(z)
        out_shapes.append(jax.ShapeDtypeStruct((1, 1), jnp.float32))
        out_specs.append(pl.BlockSpec(memory_space=pltpu.SMEM))
        body = functools.partial(_stage_body_last, nb=nb, k_sz=k_sz,
                                 inv_b=1.0 / b_sz, inv_n=1.0 / (b_sz * d),
                                 split_merge=split_merge)
    else:
        body = functools.partial(_stage_body, nb=nb, k_sz=k_sz,
                                 inv_b=1.0 / b_sz, split_merge=split_merge)
    return pl.pallas_call(
        body,
        grid=(nb,),
        in_specs=in_specs,
        out_specs=out_specs,
        out_shape=out_shapes,
        compiler_params=pltpu.CompilerParams(
            dimension_semantics=("arbitrary",)),
    )(*args)


def kernel(z, embed):
    r, k_sz, d = embed.shape
    b_sz = z.shape[0]
    bt = min(256, b_sz)
    residual = z
    zq = jnp.zeros_like(z)
    codes_list, usage_list = [], []
    loss = None
    for s in range(r):
        cb = embed[s]
        x2 = jnp.sum(residual ** 2, axis=1, keepdims=True)
        e2 = jnp.sum(cb ** 2, axis=1)[None, :]
        is_last = s == r - 1
        outs = _run_stage(x2, residual, zq, cb, e2, z, is_last, s == 0, bt)
        if is_last:
            codes_s, usage_s, residual, zq, loss = outs
        else:
            codes_s, usage_s, residual, zq = outs
        codes_list.append(codes_s)
        usage_list.append(usage_s)
    codes = jnp.concatenate(codes_list, axis=1)
    usage = jnp.concatenate(usage_list, axis=0)
    recon_loss = loss.reshape(())
    return (codes, zq, residual, recon_loss, usage)


# speedup vs baseline: 1.3827x; 1.3821x over previous
import functools

import jax
import jax.numpy as jnp
from jax.experimental import pallas as pl
from jax.experimental.pallas import tpu as pltpu


def _argmin_codes(dist, k_sz, split_merge):
    if split_merge:
        h = k_sz // 2
        d0 = dist[:, :h]
        d1 = dist[:, h:]
        m0 = jnp.min(d0, axis=1, keepdims=True)
        m1 = jnp.min(d1, axis=1, keepdims=True)
        il = jax.lax.broadcasted_iota(jnp.int32, d0.shape, 1)
        a0 = jnp.min(jnp.where(d0 == m0, il, h), axis=1)
        a1 = jnp.min(jnp.where(d1 == m1, il, h), axis=1) + h
        m0r = m0[:, 0].astype(jnp.bfloat16).astype(jnp.float32)
        return jnp.where(m1[:, 0] < m0r, a1, a0)
    m = jnp.min(dist, axis=1, keepdims=True)
    iota = jax.lax.broadcasted_iota(jnp.int32, dist.shape, 1)
    return jnp.min(jnp.where(dist == m, iota, k_sz), axis=1)


def _onehot_rows(onehot, cb):
    dot = lambda m: jax.lax.dot_general(
        onehot, m, dimension_numbers=(((1,), (0,)), ((), ())),
        preferred_element_type=jnp.float32)
    cb_hi = cb.astype(jnp.bfloat16).astype(jnp.float32)
    rem = cb - cb_hi
    cb_mid = rem.astype(jnp.bfloat16).astype(jnp.float32)
    cb_lo = rem - cb_mid
    return (dot(cb_hi) + dot(cb_mid)) + dot(cb_lo)


def _stage_body(x2_ref, res_ref, zq_ref, cb_ref, e2_ref,
                codes_ref, usage_ref, res_out, zq_out,
                *, nb, k_sz, inv_b, split_merge):
    b = pl.program_id(0)
    x = res_ref[...]
    cb = cb_ref[...]
    xe = jax.lax.dot_general(
        x, cb, dimension_numbers=(((1,), (1,)), ((), ())),
        preferred_element_type=jnp.float32)
    dist = x2_ref[...] + e2_ref[...] - 2.0 * xe
    codes = _argmin_codes(dist, k_sz, split_merge)
    iota = jax.lax.broadcasted_iota(jnp.int32, dist.shape, 1)
    onehot = (iota == codes[:, None]).astype(jnp.float32)
    quant = _onehot_rows(onehot, cb)
    counts = jnp.sum(onehot, axis=0, keepdims=True)

    codes_ref[...] = codes[:, None]
    res_out[...] = x - quant
    zq_out[...] = zq_ref[...] + quant

    @pl.when(b == 0)
    def _():
        usage_ref[...] = jnp.zeros_like(counts)

    usage_ref[...] += counts

    @pl.when(b == nb - 1)
    def _():
        usage_ref[...] = usage_ref[...] * inv_b


def _stage_body_last(x2_ref, res_ref, zq_ref, cb_ref, e2_ref, z_ref,
                     codes_ref, usage_ref, res_out, zq_out, loss_ref,
                     *, nb, k_sz, inv_b, inv_n, split_merge):
    b = pl.program_id(0)
    x = res_ref[...]
    cb = cb_ref[...]
    xe = jax.lax.dot_general(
        x, cb, dimension_numbers=(((1,), (1,)), ((), ())),
        preferred_element_type=jnp.float32)
    dist = x2_ref[...] + e2_ref[...] - 2.0 * xe
    codes = _argmin_codes(dist, k_sz, split_merge)
    iota = jax.lax.broadcasted_iota(jnp.int32, dist.shape, 1)
    onehot = (iota == codes[:, None]).astype(jnp.float32)
    quant = _onehot_rows(onehot, cb)
    counts = jnp.sum(onehot, axis=0, keepdims=True)

    codes_ref[...] = codes[:, None]
    new_res = x - quant
    new_zq = zq_ref[...] + quant
    res_out[...] = new_res
    zq_out[...] = new_zq

    @pl.when(b == 0)
    def _():
        usage_ref[...] = jnp.zeros_like(counts)
        loss_ref[0, 0] = 0.0

    usage_ref[...] += counts
    loss_ref[0, 0] += jnp.sum((new_zq - z_ref[...]) ** 2)

    @pl.when(b == nb - 1)
    def _():
        usage_ref[...] = usage_ref[...] * inv_b
        loss_ref[0, 0] = loss_ref[0, 0] * inv_n


def _run_stage(x2, residual, zq, cb, e2, z, is_last, split_merge, bt):
    b_sz, d = residual.shape
    k_sz = cb.shape[0]
    nb = b_sz // bt
    row_spec = pl.BlockSpec((bt, d), lambda b: (b, 0))
    in_specs = [
        pl.BlockSpec((bt, 1), lambda b: (b, 0)),
        row_spec,
        row_spec,
        pl.BlockSpec((k_sz, d), lambda b: (0, 0)),
        pl.BlockSpec((1, k_sz), lambda b: (0, 0)),
    ]
    out_shapes = [
        jax.ShapeDtypeStruct((b_sz, 1), jnp.int32),
        jax.ShapeDtypeStruct((1, k_sz), jnp.float32),
        jax.ShapeDtypeStruct((b_sz, d), jnp.float32),
        jax.ShapeDtypeStruct((b_sz, d), jnp.float32),
    ]
    out_specs = [
        pl.BlockSpec((bt, 1), lambda b: (b, 0)),
        pl.BlockSpec((1, k_sz), lambda b: (0, 0)),
        row_spec,
        row_spec,
    ]
    args = [x2, residual, zq, cb, e2]
    if is_last:
        in_specs.append(row_spec)
        args.append(z)
        out_shapes.append(jax.ShapeDtypeStruct((1, 1), jnp.float32))
        out_specs.append(pl.BlockSpec(memory_space=pltpu.SMEM))
        body = functools.partial(_stage_body_last, nb=nb, k_sz=k_sz,
                                 inv_b=1.0 / b_sz, inv_n=1.0 / (b_sz * d),
                                 split_merge=split_merge)
    else:
        body = functools.partial(_stage_body, nb=nb, k_sz=k_sz,
                                 inv_b=1.0 / b_sz, split_merge=split_merge)
    return pl.pallas_call(
        body,
        grid=(nb,),
        in_specs=in_specs,
        out_specs=out_specs,
        out_shape=out_shapes,
        compiler_params=pltpu.CompilerParams(
            dimension_semantics=("arbitrary",)),
    )(*args)


def kernel(z, embed):
    r, k_sz, d = embed.shape
    b_sz = z.shape[0]
    bt = min(256, b_sz)
    residual = z
    zq = jnp.zeros_like(z)
    codes_list, usage_list = [], []
    loss = None
    for s in range(r):
        cb = embed[s]
        x2 = jnp.sum(residual ** 2, axis=1, keepdims=True)
        e2 = jnp.sum(cb ** 2, axis=1)[None, :]
        is_last = s == r - 1
        outs = _run_stage(x2, residual, zq, cb, e2, z, is_last, s == 0, bt)
        if is_last:
            codes_s, usage_s, residual, zq, loss = outs
        else:
            codes_s, usage_s, residual, zq = outs
        codes_list.append(codes_s)
        usage_list.append(usage_s)
    codes = jnp.concatenate(codes_list, axis=1)
    usage = jnp.concatenate(usage_list, axis=0)
    recon_loss = loss.reshape(())
    return (codes, zq, residual, recon_loss, usage)


# speedup vs baseline: 2.3954x; 1.7323x over previous
import functools

import jax
import jax.numpy as jnp
from jax import lax
from jax.experimental import pallas as pl
from jax.experimental.pallas import tpu as pltpu
from jax.experimental.pallas import tpu_sc as plsc


def _argmin_codes(dist, k_sz, split_merge):
    if split_merge:
        h = k_sz // 2
        d0 = dist[:, :h]
        d1 = dist[:, h:]
        m0 = jnp.min(d0, axis=1, keepdims=True)
        m1 = jnp.min(d1, axis=1, keepdims=True)
        il = jax.lax.broadcasted_iota(jnp.int32, d0.shape, 1)
        a0 = jnp.min(jnp.where(d0 == m0, il, h), axis=1)
        a1 = jnp.min(jnp.where(d1 == m1, il, h), axis=1) + h
        m0r = m0[:, 0].astype(jnp.bfloat16).astype(jnp.float32)
        return jnp.where(m1[:, 0] < m0r, a1, a0)
    m = jnp.min(dist, axis=1, keepdims=True)
    iota = jax.lax.broadcasted_iota(jnp.int32, dist.shape, 1)
    return jnp.min(jnp.where(dist == m, iota, k_sz), axis=1)


def _dist_body(x2_ref, res_ref, cb_ref, e2_ref, codes_ref, usage_ref,
               *, nb, k_sz, inv_b, split_merge):
    b = pl.program_id(0)
    x = res_ref[...]
    cb = cb_ref[...]
    xe = jax.lax.dot_general(
        x, cb, dimension_numbers=(((1,), (1,)), ((), ())),
        preferred_element_type=jnp.float32)
    dist = x2_ref[...] + e2_ref[...] - 2.0 * xe
    codes = _argmin_codes(dist, k_sz, split_merge)
    iota = jax.lax.broadcasted_iota(jnp.int32, dist.shape, 1)
    onehot = (iota == codes[:, None]).astype(jnp.float32)
    counts = jnp.sum(onehot, axis=0, keepdims=True)

    codes_ref[...] = codes[:, None]

    @pl.when(b == 0)
    def _():
        usage_ref[...] = jnp.zeros_like(counts)

    usage_ref[...] += counts

    @pl.when(b == nb - 1)
    def _():
        usage_ref[...] = usage_ref[...] * inv_b


def _run_dist_stage(x2, residual, cb, e2, split_merge, bt):
    b_sz, d = residual.shape
    k_sz = cb.shape[0]
    nb = b_sz // bt
    body = functools.partial(_dist_body, nb=nb, k_sz=k_sz,
                             inv_b=1.0 / b_sz, split_merge=split_merge)
    return pl.pallas_call(
        body,
        grid=(nb,),
        in_specs=[
            pl.BlockSpec((bt, 1), lambda b: (b, 0)),
            pl.BlockSpec((bt, d), lambda b: (b, 0)),
            pl.BlockSpec((k_sz, d), lambda b: (0, 0)),
            pl.BlockSpec((1, k_sz), lambda b: (0, 0)),
        ],
        out_specs=[
            pl.BlockSpec((bt, 1), lambda b: (b, 0)),
            pl.BlockSpec((1, k_sz), lambda b: (0, 0)),
        ],
        out_shape=[
            jax.ShapeDtypeStruct((b_sz, 1), jnp.int32),
            jax.ShapeDtypeStruct((1, k_sz), jnp.float32),
        ],
        compiler_params=pltpu.CompilerParams(
            dimension_semantics=("arbitrary",)),
    )(x2, residual, cb, e2)


def _sc_gather(cb, codes_flat):
    b_sz = codes_flat.shape[0]
    d = cb.shape[1]
    info = plsc.get_sparse_core_info()
    nw = info.num_cores * info.num_subcores
    b_per_w = b_sz // nw
    mesh = plsc.VectorSubcoreMesh(core_axis_name="c", subcore_axis_name="s")

    @functools.partial(
        pl.kernel, mesh=mesh,
        out_type=jax.ShapeDtypeStruct((b_sz, d), jnp.float32),
        scratch_types=[
            pltpu.VMEM((b_per_w,), jnp.int32),
            pltpu.VMEM((b_per_w, d), jnp.float32),
            pltpu.SemaphoreType.DMA,
        ],
    )
    def k(cb_hbm, idx_hbm, out_hbm, idx_v, rows_v, sem):
        wid = lax.axis_index("s") * info.num_cores + lax.axis_index("c")
        base = wid * b_per_w
        pltpu.sync_copy(idx_hbm.at[pl.ds(base, b_per_w)], idx_v)
        pltpu.async_copy(cb_hbm.at[idx_v], rows_v, sem).wait()
        pltpu.sync_copy(rows_v, out_hbm.at[pl.ds(base, b_per_w)])

    return k(cb, codes_flat)


def _upd_body(res_ref, zq_ref, q_ref, res_out, zq_out):
    q = q_ref[...]
    res_out[...] = res_ref[...] - q
    zq_out[...] = zq_ref[...] + q


def _upd_body_last(res_ref, zq_ref, q_ref, z_ref, res_out, zq_out, loss_ref,
                   *, nb, inv_n):
    b = pl.program_id(0)
    q = q_ref[...]
    new_zq = zq_ref[...] + q
    res_out[...] = res_ref[...] - q
    zq_out[...] = new_zq

    @pl.when(b == 0)
    def _():
        loss_ref[0, 0] = 0.0

    loss_ref[0, 0] += jnp.sum((new_zq - z_ref[...]) ** 2)

    @pl.when(b == nb - 1)
    def _():
        loss_ref[0, 0] = loss_ref[0, 0] * inv_n


def _run_update(residual, zq, quant, z, is_last, bt):
    b_sz, d = residual.shape
    nb = b_sz // bt
    row_spec = pl.BlockSpec((bt, d), lambda b: (b, 0))
    in_specs = [row_spec, row_spec, row_spec]
    out_specs = [row_spec, row_spec]
    out_shapes = [
        jax.ShapeDtypeStruct((b_sz, d), jnp.float32),
        jax.ShapeDtypeStruct((b_sz, d), jnp.float32),
    ]
    args = [residual, zq, quant]
    if is_last:
        in_specs.append(row_spec)
        args.append(z)
        out_specs.append(pl.BlockSpec(memory_space=pltpu.SMEM))
        out_shapes.append(jax.ShapeDtypeStruct((1, 1), jnp.float32))
        body = functools.partial(_upd_body_last, nb=nb,
                                 inv_n=1.0 / (b_sz * d))
    else:
        body = _upd_body
    return pl.pallas_call(
        body,
        grid=(nb,),
        in_specs=in_specs,
        out_specs=out_specs,
        out_shape=out_shapes,
        compiler_params=pltpu.CompilerParams(
            dimension_semantics=("arbitrary",)),
    )(*args)


def kernel(z, embed):
    r, k_sz, d = embed.shape
    b_sz = z.shape[0]
    bt = min(256, b_sz)
    residual = z
    zq = jnp.zeros_like(z)
    codes_list, usage_list = [], []
    loss = None
    for s in range(r):
        cb = embed[s]
        x2 = jnp.sum(residual ** 2, axis=1, keepdims=True)
        e2 = jnp.sum(cb ** 2, axis=1)[None, :]
        codes_s, usage_s = _run_dist_stage(x2, residual, cb, e2, s == 0, bt)
        quant = _sc_gather(cb, codes_s.reshape(-1))
        is_last = s == r - 1
        outs = _run_update(residual, zq, quant, z, is_last, 1024)
        if is_last:
            residual, zq, loss = outs
        else:
            residual, zq = outs
        codes_list.append(codes_s)
        usage_list.append(usage_s)
    codes = jnp.concatenate(codes_list, axis=1)
    usage = jnp.concatenate(usage_list, axis=0)
    recon_loss = loss.reshape(())
    return (codes, zq, residual, recon_loss, usage)


# speedup vs baseline: 2.5280x; 1.0554x over previous
import functools

import jax
import jax.numpy as jnp
from jax import lax
from jax.experimental import pallas as pl
from jax.experimental.pallas import tpu as pltpu
from jax.experimental.pallas import tpu_sc as plsc


def _argmin_codes(dist, k_sz, split_merge):
    if split_merge:
        h = k_sz // 2
        d0 = dist[:, :h]
        d1 = dist[:, h:]
        m0 = jnp.min(d0, axis=1, keepdims=True)
        m1 = jnp.min(d1, axis=1, keepdims=True)
        il = jax.lax.broadcasted_iota(jnp.int32, d0.shape, 1)
        a0 = jnp.min(jnp.where(d0 == m0, il, h), axis=1)
        a1 = jnp.min(jnp.where(d1 == m1, il, h), axis=1) + h
        m0r = m0[:, 0].astype(jnp.bfloat16).astype(jnp.float32)
        return jnp.where(m1[:, 0] < m0r, a1, a0)
    return jnp.argmin(dist, axis=1).astype(jnp.int32)


def _dist_body(x2_ref, res_ref, cb_ref, e2_ref, codes_ref, usage_ref,
               *, nb, k_sz, inv_b, split_merge):
    b = pl.program_id(0)
    x = res_ref[...]
    cb = cb_ref[...]
    xe = jax.lax.dot_general(
        x, cb, dimension_numbers=(((1,), (1,)), ((), ())),
        preferred_element_type=jnp.float32)
    dist = x2_ref[...] + e2_ref[...] - 2.0 * xe
    codes = _argmin_codes(dist, k_sz, split_merge)
    iota = jax.lax.broadcasted_iota(jnp.int32, dist.shape, 1)
    onehot = (iota == codes[:, None]).astype(jnp.float32)
    counts = jnp.sum(onehot, axis=0, keepdims=True)

    codes_ref[...] = codes[:, None]

    @pl.when(b == 0)
    def _():
        usage_ref[...] = jnp.zeros_like(counts)

    usage_ref[...] += counts

    @pl.when(b == nb - 1)
    def _():
        usage_ref[...] = usage_ref[...] * inv_b


def _run_dist_stage(x2, residual, cb, e2, split_merge, bt):
    b_sz, d = residual.shape
    k_sz = cb.shape[0]
    nb = b_sz // bt
    body = functools.partial(_dist_body, nb=nb, k_sz=k_sz,
                             inv_b=1.0 / b_sz, split_merge=split_merge)
    return pl.pallas_call(
        body,
        grid=(nb,),
        in_specs=[
            pl.BlockSpec((bt, 1), lambda b: (b, 0)),
            pl.BlockSpec((bt, d), lambda b: (b, 0)),
            pl.BlockSpec((k_sz, d), lambda b: (0, 0)),
            pl.BlockSpec((1, k_sz), lambda b: (0, 0)),
        ],
        out_specs=[
            pl.BlockSpec((bt, 1), lambda b: (b, 0)),
            pl.BlockSpec((1, k_sz), lambda b: (0, 0)),
        ],
        out_shape=[
            jax.ShapeDtypeStruct((b_sz, 1), jnp.int32),
            jax.ShapeDtypeStruct((1, k_sz), jnp.float32),
        ],
        compiler_params=pltpu.CompilerParams(
            dimension_semantics=("arbitrary",)),
    )(x2, residual, cb, e2)


def _sc_gather(cb, codes_flat):
    b_sz = codes_flat.shape[0]
    d = cb.shape[1]
    info = plsc.get_sparse_core_info()
    nw = info.num_cores * info.num_subcores
    b_per_w = b_sz // nw
    mesh = plsc.VectorSubcoreMesh(core_axis_name="c", subcore_axis_name="s")

    @functools.partial(
        pl.kernel, mesh=mesh,
        out_type=jax.ShapeDtypeStruct((b_sz, d), jnp.float32),
        scratch_types=[
            pltpu.VMEM((b_per_w,), jnp.int32),
            pltpu.VMEM((b_per_w, d), jnp.float32),
            pltpu.SemaphoreType.DMA,
        ],
    )
    def k(cb_hbm, idx_hbm, out_hbm, idx_v, rows_v, sem):
        wid = lax.axis_index("s") * info.num_cores + lax.axis_index("c")
        base = wid * b_per_w
        pltpu.sync_copy(idx_hbm.at[pl.ds(base, b_per_w)], idx_v)
        pltpu.async_copy(cb_hbm.at[idx_v], rows_v, sem).wait()
        pltpu.sync_copy(rows_v, out_hbm.at[pl.ds(base, b_per_w)])

    return k(cb, codes_flat)


def _upd_body(res_ref, zq_ref, q_ref, res_out, zq_out):
    q = q_ref[...]
    res_out[...] = res_ref[...] - q
    zq_out[...] = zq_ref[...] + q


def _upd_body_last(res_ref, zq_ref, q_ref, z_ref, res_out, zq_out, loss_ref,
                   *, nb, inv_n):
    b = pl.program_id(0)
    q = q_ref[...]
    new_zq = zq_ref[...] + q
    res_out[...] = res_ref[...] - q
    zq_out[...] = new_zq

    @pl.when(b == 0)
    def _():
        loss_ref[0, 0] = 0.0

    loss_ref[0, 0] += jnp.sum((new_zq - z_ref[...]) ** 2)

    @pl.when(b == nb - 1)
    def _():
        loss_ref[0, 0] = loss_ref[0, 0] * inv_n


def _run_update(residual, zq, quant, z, is_last, bt):
    b_sz, d = residual.shape
    nb = b_sz // bt
    row_spec = pl.BlockSpec((bt, d), lambda b: (b, 0))
    in_specs = [row_spec, row_spec, row_spec]
    out_specs = [row_spec, row_spec]
    out_shapes = [
        jax.ShapeDtypeStruct((b_sz, d), jnp.float32),
        jax.ShapeDtypeStruct((b_sz, d), jnp.float32),
    ]
    args = [residual, zq, quant]
    if is_last:
        in_specs.append(row_spec)
        args.append(z)
        out_specs.append(pl.BlockSpec(memory_space=pltpu.SMEM))
        out_shapes.append(jax.ShapeDtypeStruct((1, 1), jnp.float32))
        body = functools.partial(_upd_body_last, nb=nb,
                                 inv_n=1.0 / (b_sz * d))
    else:
        body = _upd_body
    return pl.pallas_call(
        body,
        grid=(nb,),
        in_specs=in_specs,
        out_specs=out_specs,
        out_shape=out_shapes,
        compiler_params=pltpu.CompilerParams(
            dimension_semantics=("arbitrary",)),
    )(*args)


def kernel(z, embed):
    r, k_sz, d = embed.shape
    b_sz = z.shape[0]
    bt = min(256, b_sz)
    residual = z
    zq = jnp.zeros_like(z)
    codes_list, usage_list = [], []
    loss = None
    for s in range(r):
        cb = embed[s]
        x2 = jnp.sum(residual ** 2, axis=1, keepdims=True)
        e2 = jnp.sum(cb ** 2, axis=1)[None, :]
        codes_s, usage_s = _run_dist_stage(x2, residual, cb, e2, s == 0, bt)
        quant = _sc_gather(cb, codes_s.reshape(-1))
        is_last = s == r - 1
        outs = _run_update(residual, zq, quant, z, is_last, 1024)
        if is_last:
            residual, zq, loss = outs
        else:
            residual, zq = outs
        codes_list.append(codes_s)
        usage_list.append(usage_s)
    codes = jnp.concatenate(codes_list, axis=1)
    usage = jnp.concatenate(usage_list, axis=0)
    recon_loss = loss.reshape(())
    return (codes, zq, residual, recon_loss, usage)


# speedup vs baseline: 2.6487x; 1.0477x over previous
import functools

import jax
import jax.numpy as jnp
from jax import lax
from jax.experimental import pallas as pl
from jax.experimental.pallas import tpu as pltpu
from jax.experimental.pallas import tpu_sc as plsc


def _argmin_codes(dist, k_sz, split_merge):
    if split_merge:
        h = k_sz // 2
        d0 = dist[:, :h]
        d1 = dist[:, h:]
        m0 = jnp.min(d0, axis=1, keepdims=True)
        m1 = jnp.min(d1, axis=1, keepdims=True)
        il = jax.lax.broadcasted_iota(jnp.int32, d0.shape, 1)
        a0 = jnp.min(jnp.where(d0 == m0, il, h), axis=1)
        a1 = jnp.min(jnp.where(d1 == m1, il, h), axis=1) + h
        m0r = m0[:, 0].astype(jnp.bfloat16).astype(jnp.float32)
        return jnp.where(m1[:, 0] < m0r, a1, a0)
    return jnp.argmin(dist, axis=1).astype(jnp.int32)


def _dist_body(x2_ref, res_ref, cb_ref, e2_ref, codes_ref, usage_ref,
               *, nb, k_sz, inv_b, split_merge):
    b = pl.program_id(0)
    x = res_ref[...]
    cb = cb_ref[...]
    xe = jax.lax.dot_general(
        x, cb, dimension_numbers=(((1,), (1,)), ((), ())),
        preferred_element_type=jnp.float32)
    dist = x2_ref[...] + e2_ref[...] - 2.0 * xe
    codes = _argmin_codes(dist, k_sz, split_merge)
    iota = jax.lax.broadcasted_iota(jnp.int32, dist.shape, 1)
    onehot = (iota == codes[:, None]).astype(jnp.float32)
    counts = jnp.sum(onehot, axis=0, keepdims=True)

    codes_ref[...] = codes[:, None]

    @pl.when(b == 0)
    def _():
        usage_ref[...] = jnp.zeros_like(counts)

    usage_ref[...] += counts

    @pl.when(b == nb - 1)
    def _():
        usage_ref[...] = usage_ref[...] * inv_b


def _run_dist_stage(x2, residual, cb, e2, split_merge, bt):
    b_sz, d = residual.shape
    k_sz = cb.shape[0]
    nb = b_sz // bt
    body = functools.partial(_dist_body, nb=nb, k_sz=k_sz,
                             inv_b=1.0 / b_sz, split_merge=split_merge)
    return pl.pallas_call(
        body,
        grid=(nb,),
        in_specs=[
            pl.BlockSpec((bt, 1), lambda b: (b, 0)),
            pl.BlockSpec((bt, d), lambda b: (b, 0)),
            pl.BlockSpec((k_sz, d), lambda b: (0, 0)),
            pl.BlockSpec((1, k_sz), lambda b: (0, 0)),
        ],
        out_specs=[
            pl.BlockSpec((bt, 1), lambda b: (b, 0)),
            pl.BlockSpec((1, k_sz), lambda b: (0, 0)),
        ],
        out_shape=[
            jax.ShapeDtypeStruct((b_sz, 1), jnp.int32),
            jax.ShapeDtypeStruct((1, k_sz), jnp.float32),
        ],
        compiler_params=pltpu.CompilerParams(
            dimension_semantics=("arbitrary",)),
    )(x2, residual, cb, e2)


def _sc_gather(cb, codes_flat):
    b_sz = codes_flat.shape[0]
    d = cb.shape[1]
    info = plsc.get_sparse_core_info()
    nw = info.num_cores * info.num_subcores
    b_per_w = b_sz // nw
    mesh = plsc.VectorSubcoreMesh(core_axis_name="c", subcore_axis_name="s")

    @functools.partial(
        pl.kernel, mesh=mesh,
        out_type=jax.ShapeDtypeStruct((b_sz, d), jnp.float32),
        scratch_types=[
            pltpu.VMEM((b_per_w,), jnp.int32),
            pltpu.VMEM((b_per_w, d), jnp.float32),
            pltpu.SemaphoreType.DMA,
        ],
    )
    def k(cb_hbm, idx_hbm, out_hbm, idx_v, rows_v, sem):
        wid = lax.axis_index("s") * info.num_cores + lax.axis_index("c")
        base = wid * b_per_w
        pltpu.sync_copy(idx_hbm.at[pl.ds(base, b_per_w)], idx_v)
        pltpu.async_copy(cb_hbm.at[idx_v], rows_v, sem).wait()
        pltpu.sync_copy(rows_v, out_hbm.at[pl.ds(base, b_per_w)])

    return k(cb, codes_flat)


def _upd_body(res_ref, zq_ref, q_ref, res_out, zq_out):
    q = q_ref[...]
    res_out[...] = res_ref[...] - q
    zq_out[...] = zq_ref[...] + q


def _upd_body_last(res_ref, zq_ref, q_ref, z_ref, res_out, zq_out, loss_ref,
                   *, nb, inv_n):
    b = pl.program_id(0)
    q = q_ref[...]
    new_zq = zq_ref[...] + q
    res_out[...] = res_ref[...] - q
    zq_out[...] = new_zq

    @pl.when(b == 0)
    def _():
        loss_ref[0, 0] = 0.0

    loss_ref[0, 0] += jnp.sum((new_zq - z_ref[...]) ** 2)

    @pl.when(b == nb - 1)
    def _():
        loss_ref[0, 0] = loss_ref[0, 0] * inv_n


def _run_update(residual, zq, quant, z, is_last, bt):
    b_sz, d = residual.shape
    nb = b_sz // bt
    row_spec = pl.BlockSpec((bt, d), lambda b: (b, 0))
    in_specs = [row_spec, row_spec, row_spec]
    out_specs = [row_spec, row_spec]
    out_shapes = [
        jax.ShapeDtypeStruct((b_sz, d), jnp.float32),
        jax.ShapeDtypeStruct((b_sz, d), jnp.float32),
    ]
    args = [residual, zq, quant]
    if is_last:
        in_specs.append(row_spec)
        args.append(z)
        out_specs.append(pl.BlockSpec(memory_space=pltpu.SMEM))
        out_shapes.append(jax.ShapeDtypeStruct((1, 1), jnp.float32))
        body = functools.partial(_upd_body_last, nb=nb,
                                 inv_n=1.0 / (b_sz * d))
    else:
        body = _upd_body
    return pl.pallas_call(
        body,
        grid=(nb,),
        in_specs=in_specs,
        out_specs=out_specs,
        out_shape=out_shapes,
        compiler_params=pltpu.CompilerParams(
            dimension_semantics=("arbitrary",)),
    )(*args)


def kernel(z, embed):
    r, k_sz, d = embed.shape
    b_sz = z.shape[0]
    bt = min(512, b_sz)
    residual = z
    zq = jnp.zeros_like(z)
    codes_list, usage_list = [], []
    loss = None
    for s in range(r):
        cb = embed[s]
        x2 = jnp.sum(residual ** 2, axis=1, keepdims=True)
        e2 = jnp.sum(cb ** 2, axis=1)[None, :]
        codes_s, usage_s = _run_dist_stage(x2, residual, cb, e2, s == 0, bt)
        quant = _sc_gather(cb, codes_s.reshape(-1))
        is_last = s == r - 1
        outs = _run_update(residual, zq, quant, z, is_last, 1024)
        if is_last:
            residual, zq, loss = outs
        else:
            residual, zq = outs
        codes_list.append(codes_s)
        usage_list.append(usage_s)
    codes = jnp.concatenate(codes_list, axis=1)
    usage = jnp.concatenate(usage_list, axis=0)
    recon_loss = loss.reshape(())
    return (codes, zq, residual, recon_loss, usage)
